# baseline (speedup 1.0000x reference)
import dataclasses
import functools

import jax
import jax.numpy as jnp
from jax import lax
from jax.experimental import pallas as pl
from jax.experimental.pallas import tpu as pltpu
from jax.experimental.pallas import tpu_sc as plsc

NC = 2
NS = 16
NW = NC * NS
PAD_D = 128
BLK = 128
PAD_COLS = 2048


def _pad_body(tt_ref, o_ref):
    o_ref[:, 0:64] = tt_ref[...].T


def _pad_table(table):
    v, d = table.shape
    return pl.pallas_call(
        _pad_body,
        grid=(pl.cdiv(v, PAD_COLS),),
        in_specs=[pl.BlockSpec((d, PAD_COLS), lambda i: (0, i))],
        out_specs=pl.BlockSpec((PAD_COLS, PAD_D), lambda i: (i, 0)),
        out_shape=jax.ShapeDtypeStruct((v, PAD_D), jnp.float32),
    )(table.T)


@jax.jit
def _embed_add(table_p, z_t, u_t):
    l, b = z_t.shape
    d = u_t.shape[1]
    blocks_per_w = b // (NW * BLK)
    mesh = plsc.VectorSubcoreMesh(core_axis_name="core", subcore_axis_name="sub")

    cp = pltpu.CompilerParams()
    if "needs_layout_passes" in pltpu.CompilerParams.__dataclass_fields__:
        cp = dataclasses.replace(cp, needs_layout_passes=False)

    @functools.partial(
        pl.kernel,
        out_type=jax.ShapeDtypeStruct((l, d, b), jnp.float32),
        mesh=mesh,
        compiler_params=cp,
        scratch_types=[
            pltpu.VMEM((l, BLK), jnp.int32),
            pltpu.VMEM((l, BLK), jnp.int32),
            pltpu.VMEM((BLK, PAD_D), jnp.float32),
            pltpu.VMEM((BLK, PAD_D), jnp.float32),
            pltpu.VMEM((d, BLK), jnp.float32),
            pltpu.VMEM((d, BLK), jnp.float32),
            pltpu.VMEM((d, BLK), jnp.float32),
            pltpu.VMEM((d, BLK), jnp.float32),
        ] + [pltpu.SemaphoreType.DMA] * 8,
    )
    def k(table_hbm, z_hbm, u_hbm, out_hbm,
          idx0, idx1, gb0, gb1, ub0, ub1, ob0, ob1,
          su0, su1, sg0, sg1, so0, so1, si0, si1):
        idxs = (idx0, idx1)
        gbs = (gb0, gb1)
        ubs = (ub0, ub1)
        obs = (ob0, ob1)
        sus = (su0, su1)
        sgs = (sg0, sg1)
        sos = (so0, so1)
        sis = (si0, si1)
        wid = lax.axis_index("sub") * NC + lax.axis_index("core")
        c0w = wid * blocks_per_w * BLK

        iota16 = lax.iota(jnp.int32, 16)

        def inc(jb):
            ib = idxs[jb]
            for r in range(l):
                for w in range(0, BLK, 16):
                    ib[r, pl.ds(w, 16)] = ib[r, pl.ds(w, 16)] + 1

        def idx_copy(blk, jb):
            col0 = pl.multiple_of(c0w + blk * BLK, BLK)
            return pltpu.make_async_copy(
                z_hbm.at[:, pl.ds(col0, BLK)], idxs[jb], sis[jb])

        def u_copy(blk, li, p):
            col0 = pl.multiple_of(c0w + blk * BLK, BLK)
            return pltpu.make_async_copy(
                u_hbm.at[li].at[:, pl.ds(col0, BLK)], ubs[p], sus[p])

        def g_copy(li, p, jb):
            return pltpu.make_async_copy(
                table_hbm.at[idxs[jb].at[li]], gbs[p], sgs[p])

        def o_copy(blk, li, p):
            col0 = pl.multiple_of(c0w + blk * BLK, BLK)
            return pltpu.make_async_copy(
                obs[p], out_hbm.at[li].at[:, pl.ds(col0, BLK)], sos[p])

        def start_a(blk, li, p, jb):
            u_copy(blk, li, p).start()
            g_copy(li, p, jb).start()

        def do_b(blk, li, p, drain_pred):
            u_copy(blk, li, p).wait()
            g_copy(0, p, 0).wait()

            @pl.when(drain_pred)
            def _():
                o_copy(blk, 0, p).wait()

            @pl.loop(0, d)
            def _(dd):
                colv = jnp.full((16,), dd, jnp.int32)
                for w in range(0, BLK, 16):
                    gvals = plsc.load_gather(gbs[p], [iota16 + w, colv])
                    obs[p][dd, pl.ds(w, 16)] = (
                        ubs[p][dd, pl.ds(w, 16)] + gvals)

            o_copy(blk, li, p).start()

        c = idx_copy(0, 0)
        c.start()
        c.wait()
        inc(0)

        for blk in range(blocks_per_w):
            jb = blk % 2
            if blk + 1 < blocks_per_w:
                idx_copy(blk + 1, 1 - jb).start()

            start_a(blk, 0, 0, jb)

            @pl.loop(0, l // 2)
            def _(kk):
                li = kk * 2
                start_a(blk, li + 1, 1, jb)
                do_b(blk, li, 0, kk > 0)

                @pl.when(kk < l // 2 - 1)
                def _():
                    start_a(blk, li + 2, 0, jb)

                do_b(blk, li + 1, 1, kk > 0)

            o_copy(blk, 0, 0).wait()
            o_copy(blk, 0, 1).wait()

            if blk + 1 < blocks_per_w:
                idx_copy(blk + 1, 1 - jb).wait()
                inc(1 - jb)

    return k(table_p, z_t, u_t)


def kernel(z, u, table):
    table_p = _pad_table(table)
    z_t = z.astype(jnp.int32).T
    u_t = jnp.transpose(u, (1, 2, 0))
    out_t = _embed_add(table_p, z_t, u_t)
    v = jnp.transpose(out_t, (2, 0, 1))
    return (z, v)


# speedup vs baseline: 1.2179x; 1.2179x over previous
import functools

import jax
import jax.numpy as jnp
from jax import lax
from jax.experimental import pallas as pl
from jax.experimental.pallas import tpu as pltpu
from jax.experimental.pallas import tpu_sc as plsc

NC = 2
NS = 16
NW = NC * NS
PAD_D = 128
WB = 4
PAIR_ROWS = 8
PAD_COLS = 2048


def _pad_body(tt_ref, o_ref):
    o_ref[:, 0:64] = tt_ref[...].T


def _pad_table(table):
    v, d = table.shape
    return pl.pallas_call(
        _pad_body,
        grid=(pl.cdiv(v, PAD_COLS),),
        in_specs=[pl.BlockSpec((d, PAD_COLS), lambda i: (0, i))],
        out_specs=pl.BlockSpec((PAD_COLS, PAD_D), lambda i: (i, 0)),
        out_shape=jax.ShapeDtypeStruct((v, PAD_D), jnp.float32),
        compiler_params=pltpu.CompilerParams(
            dimension_semantics=("parallel",)),
    )(table.T)


@jax.jit
def _embed_add(table_p, z, u):
    b, l = z.shape
    d = u.shape[-1]
    b_per_w = b // NW
    n_chunks = b_per_w // WB
    n_macro = n_chunks // 4
    mesh = plsc.VectorSubcoreMesh(core_axis_name="core", subcore_axis_name="sub")

    @functools.partial(
        pl.kernel,
        out_type=jax.ShapeDtypeStruct((b, l, d), jnp.float32),
        mesh=mesh,
        scratch_types=[
            pltpu.VMEM((PAIR_ROWS, l), jnp.int32),
            pltpu.VMEM((PAIR_ROWS, l), jnp.int32),
            pltpu.VMEM((WB, l, d), jnp.float32),
            pltpu.VMEM((WB, l, d), jnp.float32),
            pltpu.VMEM((WB * l, PAD_D), jnp.float32),
            pltpu.VMEM((WB * l, PAD_D), jnp.float32),
        ] + [pltpu.SemaphoreType.DMA] * 8,
    )
    def k(table_hbm, z_hbm, u_hbm, out_hbm,
          idx0, idx1, ub0, ub1, rb0, rb1,
          su0, su1, sg0, sg1, so0, so1, si0, si1):
        idxs = (idx0, idx1)
        us = (ub0, ub1)
        rs = (rb0, rb1)
        sus = (su0, su1)
        sgs = (sg0, sg1)
        sos = (so0, so1)
        sis = (si0, si1)
        wid = lax.axis_index("sub") * NC + lax.axis_index("core")
        w0 = wid * b_per_w

        tail_inc = jnp.where(lax.iota(jnp.int32, 16) >= 14, 1, 0)

        def inc(jb):
            ib = idxs[jb]
            for r in range(PAIR_ROWS):
                for c0 in (0, 16, 32):
                    ib[r, pl.ds(c0, 16)] = ib[r, pl.ds(c0, 16)] + 1
                ib[r, pl.ds(34, 16)] = ib[r, pl.ds(34, 16)] + tail_inc

        def idx_copy(pj, jb):
            z0 = pl.multiple_of(w0 + pj * PAIR_ROWS, PAIR_ROWS)
            return pltpu.make_async_copy(
                z_hbm.at[pl.ds(z0, PAIR_ROWS)], idxs[jb], sis[jb])

        def u_copy(ci, p):
            b0 = w0 + ci * WB
            return pltpu.make_async_copy(
                u_hbm.at[pl.ds(b0, WB)], us[p], sus[p])

        def g_copy(r, p, jb, q):
            return pltpu.make_async_copy(
                table_hbm.at[idxs[jb].at[q * WB + r]],
                rs[p].at[pl.ds(r * l, l)],
                sgs[p])

        def o_copy(ci, p):
            b0 = w0 + ci * WB
            return pltpu.make_async_copy(
                us[p], out_hbm.at[pl.ds(b0, WB)], sos[p])

        def start_a(ci, p, jb, q):
            u_copy(ci, p).start()
            for r in range(WB):
                g_copy(r, p, jb, q).start()

        def do_b(ci, p):
            u_copy(ci, p).wait()
            for r in range(WB):
                g_copy(r, p, 0, 0).wait()

            @pl.loop(0, l)
            def _(li):
                for r in range(WB):
                    for c0 in (0, 16, 32, 48):
                        us[p][r, li, pl.ds(c0, 16)] = (
                            us[p][r, li, pl.ds(c0, 16)]
                            + rs[p][r * l + li, pl.ds(c0, 16)]
                        )

            o_copy(ci, p).start()

        c = idx_copy(0, 0)
        c.start()
        c.wait()
        inc(0)
        idx_copy(1, 1).start()
        start_a(0, 0, 0, 0)

        @pl.loop(0, n_macro)
        def _(mi):
            c0 = mi * 4

            @pl.when(mi > 0)
            def _():
                o_copy(0, 1).wait()

            start_a(c0 + 1, 1, 0, 1)
            do_b(c0, 0)
            idx_copy(0, 1).wait()
            inc(1)
            o_copy(0, 0).wait()
            start_a(c0 + 2, 0, 1, 0)
            do_b(c0 + 1, 1)

            @pl.when(mi < n_macro - 1)
            def _():
                idx_copy(2 * mi + 2, 0).start()

            o_copy(0, 1).wait()
            start_a(c0 + 3, 1, 1, 1)
            do_b(c0 + 2, 0)

            @pl.when(mi < n_macro - 1)
            def _():
                idx_copy(0, 0).wait()
                inc(0)
                o_copy(0, 0).wait()
                start_a(c0 + 4, 0, 0, 0)

            do_b(c0 + 3, 1)

            @pl.when(mi < n_macro - 1)
            def _():
                idx_copy(2 * mi + 3, 1).start()

        o_copy(0, 0).wait()
        o_copy(0, 1).wait()

    return k(table_p, z, u)


def kernel(z, u, table):
    table_p = _pad_table(table)
    v = _embed_add(table_p, z.astype(jnp.int32), u)
    return (z, v)
